# Initial kernel scaffold; baseline (speedup 1.0000x reference)
import jax
import jax.numpy as jnp
from jax.experimental import pallas as pl


def kernel(x, edge_index, W1, b1, W2, b2):
    raise NotImplementedError("write your pallas kernel here")


# speedup vs baseline: 72.6785x; 72.6785x over previous
import jax
import jax.numpy as jnp
from jax import lax
from jax.experimental import pallas as pl
from jax.experimental.pallas import tpu as pltpu
from jax.experimental.pallas import tpu_sc as plsc

N_NODES = 10000
N_EDGES = 320000
D_FEAT = 128
HIDDEN = 3
N_CLASSES = 10

NC = 2
NS = 16
NW = NC * NS
EPW = N_EDGES // NW
L = 16
NP = 10112
PF = NP * HIDDEN

_MESH = plsc.VectorSubcoreMesh(core_axis_name="c", subcore_axis_name="s")
_SC_PARAMS = pltpu.CompilerParams(
    use_tc_tiling_on_sc=False, needs_layout_passes=False)

def _deg_body(col_hbm, out_hbm, col_v, hist_v):
    c = lax.axis_index("c")
    s = lax.axis_index("s")
    wid = c * NS + s
    pltpu.sync_copy(col_hbm.at[pl.ds(wid * EPW, EPW)], col_v)

    def zbody(t, carry):
        hist_v[pl.ds(t * L, L)] = jnp.zeros((L,), jnp.float32)
        return carry

    lax.fori_loop(0, NP // L, zbody, 0)
    ones = jnp.ones((L,), jnp.float32)

    def body(i, carry):
        cv = col_v[pl.ds(i * L, L)]
        plsc.addupdate_scatter(hist_v, [cv], ones)
        return carry

    lax.fori_loop(0, EPW // L, body, 0)
    pltpu.sync_copy(hist_v, out_hbm.at[wid])


def _agg_body(row_hbm, col_hbm, p_hbm, out_hbm, row_v, col_v, p_v, acc_v):
    c = lax.axis_index("c")
    s = lax.axis_index("s")
    wid = c * NS + s
    pltpu.sync_copy(row_hbm.at[pl.ds(wid * EPW, EPW)], row_v)
    pltpu.sync_copy(col_hbm.at[pl.ds(wid * EPW, EPW)], col_v)
    pltpu.sync_copy(p_hbm, p_v)

    def zbody(t, carry):
        acc_v[pl.ds(t * L, L)] = jnp.zeros((L,), jnp.float32)
        return carry

    lax.fori_loop(0, PF // L, zbody, 0)

    def body(i, carry):
        rv = row_v[pl.ds(i * L, L)]
        cv = col_v[pl.ds(i * L, L)]
        for j in range(HIDDEN):
            g = plsc.load_gather(p_v, [rv + j * NP])
            plsc.addupdate_scatter(acc_v, [cv + j * NP], g)
        return carry

    lax.fori_loop(0, EPW // L, body, 0)
    pltpu.sync_copy(acc_v, out_hbm.at[wid])


def _dense1_body(x_ref, w1_ref, hist_ref, pt_ref, dinv_ref):
    h = jnp.dot(x_ref[...], w1_ref[...], preferred_element_type=jnp.float32)
    ht = h.T
    deg = jnp.sum(hist_ref[...], axis=0, keepdims=True) + 1.0
    dinv = lax.rsqrt(deg)
    pt_ref[...] = ht * dinv
    dinv_ref[...] = dinv


def _dense2_body(acc_ref, pt_ref, dinv_ref, b1_ref, w2_ref, b2_ref, h_ref,
                 z_ref):
    acc = jnp.sum(acc_ref[...], axis=0)
    t = (acc + pt_ref[...]) * dinv_ref[...] + b1_ref[...]
    hrt = jnp.maximum(t, 0.0)
    hr = hrt.T
    h_ref[...] = hr
    z_ref[...] = jnp.dot(hr, w2_ref[...],
                         preferred_element_type=jnp.float32) + b2_ref[...]


_deg_kernel = pl.kernel(
    _deg_body,
    out_type=jax.ShapeDtypeStruct((NW, NP), jnp.float32),
    mesh=_MESH,
    scratch_types=[
        pltpu.VMEM((EPW,), jnp.int32),
        pltpu.VMEM((NP,), jnp.float32),
    ],
    compiler_params=_SC_PARAMS,
)

_agg_kernel = pl.kernel(
    _agg_body,
    out_type=jax.ShapeDtypeStruct((NW, PF), jnp.float32),
    mesh=_MESH,
    scratch_types=[
        pltpu.VMEM((EPW,), jnp.int32),
        pltpu.VMEM((EPW,), jnp.int32),
        pltpu.VMEM((PF,), jnp.float32),
        pltpu.VMEM((PF,), jnp.float32),
    ],
    compiler_params=_SC_PARAMS,
)

_dense1_kernel = pl.pallas_call(
    _dense1_body,
    out_shape=[
        jax.ShapeDtypeStruct((HIDDEN, NP), jnp.float32),
        jax.ShapeDtypeStruct((1, NP), jnp.float32),
    ],
)

_dense2_kernel = pl.pallas_call(
    _dense2_body,
    out_shape=[
        jax.ShapeDtypeStruct((NP, HIDDEN), jnp.float32),
        jax.ShapeDtypeStruct((NP, N_CLASSES), jnp.float32),
    ],
)


def kernel(x, edge_index, W1, b1, W2, b2):
    ei = edge_index.astype(jnp.int32)
    row = ei[0]
    col = ei[1]
    x_pad = jnp.zeros((NP, D_FEAT), jnp.float32).at[:N_NODES].set(x)

    hist = _deg_kernel(col)
    pt, dinv = _dense1_kernel(x_pad, W1, hist)
    acc = _agg_kernel(row, col, pt.reshape(PF))
    h, z = _dense2_kernel(acc.reshape(NW, HIDDEN, NP), pt, dinv,
                          b1.reshape(HIDDEN, 1), W2, b2.reshape(1, N_CLASSES))
    return (h[:N_NODES], z[:N_NODES])


# speedup vs baseline: 80.1071x; 1.1022x over previous
import jax
import jax.numpy as jnp
from jax import lax
from jax.experimental import pallas as pl
from jax.experimental.pallas import tpu as pltpu
from jax.experimental.pallas import tpu_sc as plsc

N_NODES = 10000
N_EDGES = 320000
D_FEAT = 128
HIDDEN = 3
N_CLASSES = 10

NC = 2
NS = 16
NW = NC * NS
EPW = N_EDGES // NW
L = 16
NP = N_NODES
PF = NP * HIDDEN

_MESH = plsc.VectorSubcoreMesh(core_axis_name="c", subcore_axis_name="s")
_SC_PARAMS = pltpu.CompilerParams(
    use_tc_tiling_on_sc=False, needs_layout_passes=False)

def _deg_body(col_hbm, out_hbm, col_v, hist_v):
    c = lax.axis_index("c")
    s = lax.axis_index("s")
    wid = c * NS + s
    pltpu.sync_copy(col_hbm.at[pl.ds(wid * EPW, EPW)], col_v)

    def zbody(t, carry):
        hist_v[pl.ds(t * L, L)] = jnp.zeros((L,), jnp.float32)
        return carry

    lax.fori_loop(0, NP // L, zbody, 0, unroll=8)
    ones = jnp.ones((L,), jnp.float32)

    def body(i, carry):
        cv = col_v[pl.ds(i * L, L)]
        plsc.addupdate_scatter(hist_v, [cv], ones)
        return carry

    lax.fori_loop(0, EPW // L, body, 0, unroll=8)
    pltpu.sync_copy(hist_v, out_hbm.at[wid])


def _agg_body(row_hbm, col_hbm, p_hbm, out_hbm, row_v, col_v, p_v, acc_v):
    c = lax.axis_index("c")
    s = lax.axis_index("s")
    wid = c * NS + s
    pltpu.sync_copy(row_hbm.at[pl.ds(wid * EPW, EPW)], row_v)
    pltpu.sync_copy(col_hbm.at[pl.ds(wid * EPW, EPW)], col_v)
    pltpu.sync_copy(p_hbm, p_v)

    def zbody(t, carry):
        acc_v[pl.ds(t * L, L)] = jnp.zeros((L,), jnp.float32)
        return carry

    lax.fori_loop(0, PF // L, zbody, 0, unroll=8)

    def body(i, carry):
        rv = row_v[pl.ds(i * L, L)]
        cv = col_v[pl.ds(i * L, L)]
        for j in range(HIDDEN):
            g = plsc.load_gather(p_v, [rv + j * NP])
            plsc.addupdate_scatter(acc_v, [cv + j * NP], g)
        return carry

    lax.fori_loop(0, EPW // L, body, 0, unroll=4)
    pltpu.sync_copy(acc_v, out_hbm.at[wid])


def _dense1_body(x_ref, w1_ref, hist_ref, pt_ref, dinv_ref):
    h = jnp.dot(x_ref[...], w1_ref[...], preferred_element_type=jnp.float32)
    ht = h.T
    deg = jnp.sum(hist_ref[...], axis=0, keepdims=True) + 1.0
    dinv = lax.rsqrt(deg)
    pt_ref[...] = ht * dinv
    dinv_ref[...] = dinv


def _dense2_body(acc_ref, pt_ref, dinv_ref, b1_ref, w2_ref, b2_ref, h_ref,
                 z_ref):
    acc = jnp.sum(acc_ref[...], axis=0)
    t = (acc + pt_ref[...]) * dinv_ref[...] + b1_ref[...]
    hrt = jnp.maximum(t, 0.0)
    hr = hrt.T
    h_ref[...] = hr
    z_ref[...] = jnp.dot(hr, w2_ref[...],
                         preferred_element_type=jnp.float32) + b2_ref[...]


_deg_kernel = pl.kernel(
    _deg_body,
    out_type=jax.ShapeDtypeStruct((NW, NP), jnp.float32),
    mesh=_MESH,
    scratch_types=[
        pltpu.VMEM((EPW,), jnp.int32),
        pltpu.VMEM((NP,), jnp.float32),
    ],
    compiler_params=_SC_PARAMS,
)

_agg_kernel = pl.kernel(
    _agg_body,
    out_type=jax.ShapeDtypeStruct((NW, PF), jnp.float32),
    mesh=_MESH,
    scratch_types=[
        pltpu.VMEM((EPW,), jnp.int32),
        pltpu.VMEM((EPW,), jnp.int32),
        pltpu.VMEM((PF,), jnp.float32),
        pltpu.VMEM((PF,), jnp.float32),
    ],
    compiler_params=_SC_PARAMS,
)

_dense1_kernel = pl.pallas_call(
    _dense1_body,
    out_shape=[
        jax.ShapeDtypeStruct((HIDDEN, NP), jnp.float32),
        jax.ShapeDtypeStruct((1, NP), jnp.float32),
    ],
)

_dense2_kernel = pl.pallas_call(
    _dense2_body,
    out_shape=[
        jax.ShapeDtypeStruct((NP, HIDDEN), jnp.float32),
        jax.ShapeDtypeStruct((NP, N_CLASSES), jnp.float32),
    ],
)


def kernel(x, edge_index, W1, b1, W2, b2):
    ei = edge_index.astype(jnp.int32)
    row = ei[0]
    col = ei[1]

    hist = _deg_kernel(col)
    pt, dinv = _dense1_kernel(x, W1, hist)
    acc = _agg_kernel(row, col, pt.reshape(PF))
    h, z = _dense2_kernel(acc.reshape(NW, HIDDEN, NP), pt, dinv,
                          b1.reshape(HIDDEN, 1), W2, b2.reshape(1, N_CLASSES))
    return (h, z)


# speedup vs baseline: 93.0681x; 1.1618x over previous
import jax
import jax.numpy as jnp
from jax import lax
from jax.experimental import pallas as pl
from jax.experimental.pallas import tpu as pltpu
from jax.experimental.pallas import tpu_sc as plsc

N_NODES = 10000
N_EDGES = 320000
D_FEAT = 128
HIDDEN = 3
N_CLASSES = 10

NC = 2
NS = 16
NW = NC * NS
EPW = N_EDGES // NW
L = 16
NP = N_NODES
PF = NP * HIDDEN

_MESH = plsc.VectorSubcoreMesh(core_axis_name="c", subcore_axis_name="s")
_SC_PARAMS = pltpu.CompilerParams(
    use_tc_tiling_on_sc=False, needs_layout_passes=False)

def _deg_body(ei_hbm, out_hbm, col_v, hist_v):
    c = lax.axis_index("c")
    s = lax.axis_index("s")
    wid = c * NS + s
    pltpu.sync_copy(ei_hbm.at[1, pl.ds(wid * EPW, EPW)], col_v)

    def zbody(t, carry):
        hist_v[pl.ds(t * L, L)] = jnp.zeros((L,), jnp.float32)
        return carry

    lax.fori_loop(0, NP // L, zbody, 0, unroll=8)
    ones = jnp.ones((L,), jnp.float32)

    def body(i, carry):
        cv = col_v[pl.ds(i * L, L)]
        plsc.addupdate_scatter(hist_v, [cv], ones)
        return carry

    lax.fori_loop(0, EPW // L, body, 0, unroll=8)
    pltpu.sync_copy(hist_v, out_hbm.at[wid])


def _agg_body(ei_hbm, p_hbm, out_hbm, row_v, col_v, p_v, acc_v):
    c = lax.axis_index("c")
    s = lax.axis_index("s")
    wid = c * NS + s
    pltpu.sync_copy(ei_hbm.at[0, pl.ds(wid * EPW, EPW)], row_v)
    pltpu.sync_copy(ei_hbm.at[1, pl.ds(wid * EPW, EPW)], col_v)
    for j in range(HIDDEN):
        pltpu.sync_copy(p_hbm.at[j], p_v.at[pl.ds(j * NP, NP)])

    def zbody(t, carry):
        acc_v[pl.ds(t * L, L)] = jnp.zeros((L,), jnp.float32)
        return carry

    lax.fori_loop(0, PF // L, zbody, 0, unroll=8)

    def body(i, carry):
        rv = row_v[pl.ds(i * L, L)]
        cv = col_v[pl.ds(i * L, L)]
        for j in range(HIDDEN):
            g = plsc.load_gather(p_v, [rv + j * NP])
            plsc.addupdate_scatter(acc_v, [cv + j * NP], g)
        return carry

    lax.fori_loop(0, EPW // L, body, 0, unroll=8)
    for j in range(HIDDEN):
        pltpu.sync_copy(acc_v.at[pl.ds(j * NP, NP)],
                        out_hbm.at[wid * HIDDEN + j])


def _dense1_body(x_ref, w1_ref, hist_ref, pt_ref, dinv_ref):
    h = jnp.dot(x_ref[...], w1_ref[...], preferred_element_type=jnp.float32)
    ht = h.T
    deg = jnp.sum(hist_ref[...], axis=0, keepdims=True) + 1.0
    dinv = lax.rsqrt(deg)
    pt_ref[...] = ht * dinv
    dinv_ref[...] = dinv


def _dense2_body(acc_ref, pt_ref, dinv_ref, b1_ref, w2_ref, b2_ref, h_ref,
                 z_ref):
    a = acc_ref[...]
    acc = a[0:HIDDEN]
    for w in range(1, NW):
        acc = acc + a[w * HIDDEN:(w + 1) * HIDDEN]
    t = (acc + pt_ref[...]) * dinv_ref[...] + b1_ref[...]
    hrt = jnp.maximum(t, 0.0)
    hr = hrt.T
    h_ref[...] = hr
    z_ref[...] = jnp.dot(hr, w2_ref[...],
                         preferred_element_type=jnp.float32) + b2_ref[...]


_deg_kernel = pl.kernel(
    _deg_body,
    out_type=jax.ShapeDtypeStruct((NW, NP), jnp.float32),
    mesh=_MESH,
    scratch_types=[
        pltpu.VMEM((EPW,), jnp.int32),
        pltpu.VMEM((NP,), jnp.float32),
    ],
    compiler_params=_SC_PARAMS,
)

_agg_kernel = pl.kernel(
    _agg_body,
    out_type=jax.ShapeDtypeStruct((NW * HIDDEN, NP), jnp.float32),
    mesh=_MESH,
    scratch_types=[
        pltpu.VMEM((EPW,), jnp.int32),
        pltpu.VMEM((EPW,), jnp.int32),
        pltpu.VMEM((PF,), jnp.float32),
        pltpu.VMEM((PF,), jnp.float32),
    ],
    compiler_params=_SC_PARAMS,
)

_dense1_kernel = pl.pallas_call(
    _dense1_body,
    out_shape=[
        jax.ShapeDtypeStruct((HIDDEN, NP), jnp.float32),
        jax.ShapeDtypeStruct((1, NP), jnp.float32),
    ],
)

_dense2_kernel = pl.pallas_call(
    _dense2_body,
    out_shape=[
        jax.ShapeDtypeStruct((NP, HIDDEN), jnp.float32),
        jax.ShapeDtypeStruct((NP, N_CLASSES), jnp.float32),
    ],
)


def kernel(x, edge_index, W1, b1, W2, b2):
    ei = edge_index.astype(jnp.int32)

    hist = _deg_kernel(ei)
    pt, dinv = _dense1_kernel(x, W1, hist)
    acc = _agg_kernel(ei, pt)
    h, z = _dense2_kernel(acc, pt, dinv, b1.reshape(HIDDEN, 1), W2,
                          b2.reshape(1, N_CLASSES))
    return (h, z)


# speedup vs baseline: 123.7474x; 1.3296x over previous
import jax
import jax.numpy as jnp
from jax import lax
from jax.experimental import pallas as pl
from jax.experimental.pallas import tpu as pltpu
from jax.experimental.pallas import tpu_sc as plsc

N_NODES = 10000
N_EDGES = 320000
D_FEAT = 128
HIDDEN = 3
N_CLASSES = 10

NC = 2
NS = 16
NW = NC * NS
EPW = N_EDGES // NW
L = 16
NP = N_NODES
PF = NP * HIDDEN

_MESH = plsc.VectorSubcoreMesh(core_axis_name="c", subcore_axis_name="s")
_SC_PARAMS = pltpu.CompilerParams(
    use_tc_tiling_on_sc=False, needs_layout_passes=False)

def _deg_body(ei_hbm, out_hbm, col_v, hist_v):
    c = lax.axis_index("c")
    s = lax.axis_index("s")
    wid = c * NS + s
    pltpu.sync_copy(ei_hbm.at[1, pl.ds(wid * EPW, EPW)], col_v)

    def zbody(t, carry):
        hist_v[pl.ds(t * L, L)] = jnp.zeros((L,), jnp.float32)
        return carry

    lax.fori_loop(0, NP // L, zbody, 0, unroll=8)
    ones = jnp.ones((L,), jnp.float32)

    @plsc.parallel_loop(0, EPW // L, unroll=8)
    def _hist_loop(i):
        cv = col_v[pl.ds(i * L, L)]
        plsc.addupdate_scatter(hist_v, [cv], ones)
    pltpu.sync_copy(hist_v, out_hbm.at[wid])


def _agg_body(ei_hbm, p_hbm, out_hbm, row_v, col_v, p_v, acc_v):
    c = lax.axis_index("c")
    s = lax.axis_index("s")
    wid = c * NS + s
    pltpu.sync_copy(ei_hbm.at[0, pl.ds(wid * EPW, EPW)], row_v)
    pltpu.sync_copy(ei_hbm.at[1, pl.ds(wid * EPW, EPW)], col_v)
    for j in range(HIDDEN):
        pltpu.sync_copy(p_hbm.at[j], p_v.at[pl.ds(j * NP, NP)])

    def zbody(t, carry):
        acc_v[pl.ds(t * L, L)] = jnp.zeros((L,), jnp.float32)
        return carry

    lax.fori_loop(0, PF // L, zbody, 0, unroll=8)

    @plsc.parallel_loop(0, EPW // L, unroll=8)
    def _agg_loop(i):
        rv = row_v[pl.ds(i * L, L)]
        cv = col_v[pl.ds(i * L, L)]
        for j in range(HIDDEN):
            g = plsc.load_gather(p_v, [rv + j * NP])
            plsc.addupdate_scatter(acc_v, [cv + j * NP], g)
    for j in range(HIDDEN):
        pltpu.sync_copy(acc_v.at[pl.ds(j * NP, NP)],
                        out_hbm.at[wid * HIDDEN + j])


def _dense1_body(x_ref, w1_ref, hist_ref, pt_ref, dinv_ref):
    h = jnp.dot(x_ref[...], w1_ref[...], preferred_element_type=jnp.float32)
    ht = h.T
    deg = jnp.sum(hist_ref[...], axis=0, keepdims=True) + 1.0
    dinv = lax.rsqrt(deg)
    pt_ref[...] = ht * dinv
    dinv_ref[...] = dinv


def _dense2_body(acc_ref, pt_ref, dinv_ref, b1_ref, w2t_ref, b2_ref, ht_ref,
                 zt_ref):
    a = acc_ref[...]
    acc = a[0:HIDDEN]
    for w in range(1, NW):
        acc = acc + a[w * HIDDEN:(w + 1) * HIDDEN]
    t = (acc + pt_ref[...]) * dinv_ref[...] + b1_ref[...]
    hrt = jnp.maximum(t, 0.0)
    ht_ref[...] = hrt
    zt_ref[...] = jnp.dot(w2t_ref[...], hrt,
                          preferred_element_type=jnp.float32) + b2_ref[...]


_deg_kernel = pl.kernel(
    _deg_body,
    out_type=jax.ShapeDtypeStruct((NW, NP), jnp.float32),
    mesh=_MESH,
    scratch_types=[
        pltpu.VMEM((EPW,), jnp.int32),
        pltpu.VMEM((NP,), jnp.float32),
    ],
    compiler_params=_SC_PARAMS,
)

_agg_kernel = pl.kernel(
    _agg_body,
    out_type=jax.ShapeDtypeStruct((NW * HIDDEN, NP), jnp.float32),
    mesh=_MESH,
    scratch_types=[
        pltpu.VMEM((EPW,), jnp.int32),
        pltpu.VMEM((EPW,), jnp.int32),
        pltpu.VMEM((PF,), jnp.float32),
        pltpu.VMEM((PF,), jnp.float32),
    ],
    compiler_params=_SC_PARAMS,
)

_dense1_kernel = pl.pallas_call(
    _dense1_body,
    out_shape=[
        jax.ShapeDtypeStruct((HIDDEN, NP), jnp.float32),
        jax.ShapeDtypeStruct((1, NP), jnp.float32),
    ],
)

_dense2_kernel = pl.pallas_call(
    _dense2_body,
    out_shape=[
        jax.ShapeDtypeStruct((HIDDEN, NP), jnp.float32),
        jax.ShapeDtypeStruct((N_CLASSES, NP), jnp.float32),
    ],
)


def kernel(x, edge_index, W1, b1, W2, b2):
    ei = edge_index.astype(jnp.int32)

    hist = _deg_kernel(ei)
    pt, dinv = _dense1_kernel(x, W1, hist)
    acc = _agg_kernel(ei, pt)
    ht, zt = _dense2_kernel(acc, pt, dinv, b1.reshape(HIDDEN, 1), W2.T,
                            b2.reshape(N_CLASSES, 1))
    return (ht.T, zt.T)


# speedup vs baseline: 151.0263x; 1.2204x over previous
import jax
import jax.numpy as jnp
from jax import lax
from jax.experimental import pallas as pl
from jax.experimental.pallas import tpu as pltpu
from jax.experimental.pallas import tpu_sc as plsc

N_NODES = 10000
N_EDGES = 320000
D_FEAT = 128
HIDDEN = 3
N_CLASSES = 10

NC = 2
NS = 16
NW = NC * NS
EPW = N_EDGES // NW
L = 16
NP = N_NODES
PF = NP * HIDDEN

_MESH = plsc.VectorSubcoreMesh(core_axis_name="c", subcore_axis_name="s")
_SC_PARAMS = pltpu.CompilerParams(
    use_tc_tiling_on_sc=False, needs_layout_passes=False)


def _deg_body(ei_hbm, out_hbm, col_v, hist_v):
    c = lax.axis_index("c")
    s = lax.axis_index("s")
    wid = c * NS + s
    pltpu.sync_copy(ei_hbm.at[1, pl.ds(wid * EPW, EPW)], col_v)

    def zbody(t, carry):
        hist_v[pl.ds(t * L, L)] = jnp.zeros((L,), jnp.float32)
        return carry

    lax.fori_loop(0, NP // L, zbody, 0, unroll=8)
    ones = jnp.ones((L,), jnp.float32)

    @plsc.parallel_loop(0, EPW // L, unroll=8)
    def _hist_loop(i):
        cv = col_v[pl.ds(i * L, L)]
        plsc.addupdate_scatter(hist_v, [cv], ones)

    pltpu.sync_copy(hist_v, out_hbm.at[pl.ds(wid * NP, NP)])


def _agg_body(ei_hbm, p_hbm, out_hbm, row_v, col_v, p_v, acc_v):
    c = lax.axis_index("c")
    s = lax.axis_index("s")
    wid = c * NS + s
    pltpu.sync_copy(ei_hbm.at[0, pl.ds(wid * EPW, EPW)], row_v)
    pltpu.sync_copy(ei_hbm.at[1, pl.ds(wid * EPW, EPW)], col_v)
    pltpu.sync_copy(p_hbm, p_v)

    def zbody(t, carry):
        acc_v[pl.ds(t * L, L)] = jnp.zeros((L,), jnp.float32)
        return carry

    lax.fori_loop(0, PF // L, zbody, 0, unroll=8)

    @plsc.parallel_loop(0, EPW // L, unroll=8)
    def _agg_loop(i):
        rv = row_v[pl.ds(i * L, L)]
        cv = col_v[pl.ds(i * L, L)]
        for j in range(HIDDEN):
            g = plsc.load_gather(p_v, [rv + j * NP])
            plsc.addupdate_scatter(acc_v, [cv + j * NP], g)

    pltpu.sync_copy(acc_v, out_hbm.at[pl.ds(wid * PF, PF)])


def _mm1_body(x_ref, w1_ref, ht_ref):
    h = jnp.dot(x_ref[...], w1_ref[...], preferred_element_type=jnp.float32)
    ht_ref[...] = h.T


def _scale_body(ht_ref, hist_ref, p_ref, dinv_ref):
    deg = hist_ref[pl.ds(0, NP)]
    for w in range(1, NW):
        deg = deg + hist_ref[pl.ds(w * NP, NP)]
    dinv = lax.rsqrt(deg + 1.0)
    dinv_ref[...] = dinv
    for j in range(HIDDEN):
        p_ref[pl.ds(j * NP, NP)] = ht_ref[j] * dinv


def _dense2_body(acc_ref, p_ref, dinv_ref, b1_ref, w2t_ref, b2_ref, ht_ref,
                 zt_ref):
    dinv = dinv_ref[...]
    rows = []
    for j in range(HIDDEN):
        a = acc_ref[pl.ds(j * NP, NP)]
        for w in range(1, NW):
            a = a + acc_ref[pl.ds((w * HIDDEN + j) * NP, NP)]
        t = (a + p_ref[pl.ds(j * NP, NP)]) * dinv + b1_ref[j]
        rows.append(jnp.maximum(t, 0.0))
    hrt = jnp.stack(rows)
    ht_ref[...] = hrt
    zt_ref[...] = jnp.dot(w2t_ref[...], hrt,
                          preferred_element_type=jnp.float32) + b2_ref[...]


_deg_kernel = pl.kernel(
    _deg_body,
    out_type=jax.ShapeDtypeStruct((NW * NP,), jnp.float32),
    mesh=_MESH,
    scratch_types=[
        pltpu.VMEM((EPW,), jnp.int32),
        pltpu.VMEM((NP,), jnp.float32),
    ],
    compiler_params=_SC_PARAMS,
)

_agg_kernel = pl.kernel(
    _agg_body,
    out_type=jax.ShapeDtypeStruct((NW * PF,), jnp.float32),
    mesh=_MESH,
    scratch_types=[
        pltpu.VMEM((EPW,), jnp.int32),
        pltpu.VMEM((EPW,), jnp.int32),
        pltpu.VMEM((PF,), jnp.float32),
        pltpu.VMEM((PF,), jnp.float32),
    ],
    compiler_params=_SC_PARAMS,
)

_mm1_kernel = pl.pallas_call(
    _mm1_body,
    out_shape=jax.ShapeDtypeStruct((HIDDEN, NP), jnp.float32),
)

_scale_kernel = pl.pallas_call(
    _scale_body,
    out_shape=[
        jax.ShapeDtypeStruct((PF,), jnp.float32),
        jax.ShapeDtypeStruct((NP,), jnp.float32),
    ],
)

_dense2_kernel = pl.pallas_call(
    _dense2_body,
    out_shape=[
        jax.ShapeDtypeStruct((HIDDEN, NP), jnp.float32),
        jax.ShapeDtypeStruct((N_CLASSES, NP), jnp.float32),
    ],
)


def kernel(x, edge_index, W1, b1, W2, b2):
    ei = edge_index.astype(jnp.int32)

    hist = _deg_kernel(ei)
    ht = _mm1_kernel(x, W1)
    p, dinv = _scale_kernel(ht, hist)
    acc = _agg_kernel(ei, p)
    ht_out, zt = _dense2_kernel(acc, p, dinv, b1, W2.T,
                                b2.reshape(N_CLASSES, 1))
    return (ht_out.T, zt.T)
